# Initial kernel scaffold; baseline (speedup 1.0000x reference)
import jax
import jax.numpy as jnp
from jax.experimental import pallas as pl


def kernel(hidden_states, gate_W, e_score_correction_bias, We_gate_up, We_down, Ws_gate_up, Ws_down):
    raise NotImplementedError("write your pallas kernel here")


# speedup vs baseline: 1.5164x; 1.5164x over previous
import functools

import jax
import jax.numpy as jnp
from jax.experimental import pallas as pl
from jax.experimental.pallas import tpu as pltpu

T = 2048
H = 768
E = 8
TOPK = 2
DFF = 384
NG = 4
TG = 2
RSF = 2.5

BT = 256


def _topk_mask(x, k):
    n = x.shape[1]
    rank_cols = []
    for j in range(n):
        xj = x[:, j:j + 1]
        beats = (x > xj).astype(jnp.float32)
        if j > 0:
            ties = (x[:, :j] == xj).astype(jnp.float32)
            rank_j = (jnp.sum(beats, axis=1, keepdims=True)
                      + jnp.sum(ties, axis=1, keepdims=True))
        else:
            rank_j = jnp.sum(beats, axis=1, keepdims=True)
        rank_cols.append(rank_j)
    rank = jnp.concatenate(rank_cols, axis=1)
    return (rank < k).astype(jnp.float32)


def _routing_weights(logits, bias):
    scores = jax.nn.sigmoid(logits)
    s = scores + bias
    g = jnp.concatenate(
        [s[:, 2 * j:2 * j + 1] + s[:, 2 * j + 1:2 * j + 2]
         for j in range(NG)], axis=1)
    sel_g = _topk_mask(g, TG)
    mask_e = jnp.concatenate(
        [sel_g[:, e // (E // NG):e // (E // NG) + 1] for e in range(E)],
        axis=1)
    tmp = jnp.where(mask_e > 0.5, s, 0.0)
    sel_e = _topk_mask(tmp, TOPK)
    w = scores * sel_e
    return w / jnp.sum(w, axis=-1, keepdims=True)


def _moe_body(x_ref, gw_ref, bias_ref, wgu_ref, wd_ref, sgu_ref, sd_ref,
              out_ref):
    x = x_ref[...]
    logits = jax.lax.dot_general(
        x, gw_ref[...], (((1,), (1,)), ((), ())),
        preferred_element_type=jnp.float32)
    w_full = _routing_weights(logits, bias_ref[...])

    xb = x.astype(jnp.bfloat16)
    acc = jnp.zeros((BT, H), jnp.float32)
    for e in range(E):
        gu = jax.lax.dot_general(
            xb, wgu_ref[e], (((1,), (1,)), ((), ())),
            preferred_element_type=jnp.float32)
        gate = gu[:, :DFF]
        up = gu[:, DFF:]
        h = (gate * jax.nn.sigmoid(gate) * up).astype(jnp.bfloat16)
        d = jax.lax.dot_general(
            h, wd_ref[e], (((1,), (1,)), ((), ())),
            preferred_element_type=jnp.float32)
        acc = acc + d * w_full[:, e][:, None]

    sgu = jax.lax.dot_general(
        xb, sgu_ref[...], (((1,), (1,)), ((), ())),
        preferred_element_type=jnp.float32)
    sgate = sgu[:, :DFF]
    sup = sgu[:, DFF:]
    sh = (sgate * jax.nn.sigmoid(sgate) * sup).astype(jnp.bfloat16)
    shared = jax.lax.dot_general(
        sh, sd_ref[...], (((1,), (1,)), ((), ())),
        preferred_element_type=jnp.float32)

    out_ref[...] = acc * RSF + shared


@jax.jit
def kernel(hidden_states, gate_W, e_score_correction_bias, We_gate_up,
           We_down, Ws_gate_up, Ws_down):
    bias2d = e_score_correction_bias.reshape(1, E)
    wgu = We_gate_up.astype(jnp.bfloat16)
    wd = We_down.astype(jnp.bfloat16)
    sgu = Ws_gate_up.astype(jnp.bfloat16)
    sd = Ws_down.astype(jnp.bfloat16)

    grid = (T // BT,)
    return pl.pallas_call(
        _moe_body,
        grid=grid,
        in_specs=[
            pl.BlockSpec((BT, H), lambda i: (i, 0)),
            pl.BlockSpec((E, H), lambda i: (0, 0)),
            pl.BlockSpec((1, E), lambda i: (0, 0)),
            pl.BlockSpec((E, 2 * DFF, H), lambda i: (0, 0, 0)),
            pl.BlockSpec((E, H, DFF), lambda i: (0, 0, 0)),
            pl.BlockSpec((2 * DFF, H), lambda i: (0, 0)),
            pl.BlockSpec((H, DFF), lambda i: (0, 0)),
        ],
        out_specs=pl.BlockSpec((BT, H), lambda i: (i, 0)),
        out_shape=jax.ShapeDtypeStruct((T, H), jnp.float32),
        compiler_params=pltpu.CompilerParams(
            dimension_semantics=("arbitrary",),
        ),
    )(hidden_states, gate_W, bias2d, wgu, wd, sgu, sd)


# speedup vs baseline: 1.6316x; 1.0760x over previous
import functools

import jax
import jax.numpy as jnp
from jax.experimental import pallas as pl
from jax.experimental.pallas import tpu as pltpu

T = 2048
H = 768
E = 8
TOPK = 2
DFF = 384
NG = 4
TG = 2
RSF = 2.5

BT = 256


def _topk_mask(x, k):
    n = x.shape[1]
    rank_cols = []
    for j in range(n):
        xj = x[:, j:j + 1]
        beats = (x > xj).astype(jnp.float32)
        if j > 0:
            ties = (x[:, :j] == xj).astype(jnp.float32)
            rank_j = (jnp.sum(beats, axis=1, keepdims=True)
                      + jnp.sum(ties, axis=1, keepdims=True))
        else:
            rank_j = jnp.sum(beats, axis=1, keepdims=True)
        rank_cols.append(rank_j)
    rank = jnp.concatenate(rank_cols, axis=1)
    return (rank < k).astype(jnp.float32)


def _routing_weights(logits, bias):
    scores = jax.nn.sigmoid(logits)
    s = scores + bias
    g = jnp.concatenate(
        [s[:, 2 * j:2 * j + 1] + s[:, 2 * j + 1:2 * j + 2]
         for j in range(NG)], axis=1)
    sel_g = _topk_mask(g, TG)
    mask_e = jnp.concatenate(
        [sel_g[:, e // (E // NG):e // (E // NG) + 1] for e in range(E)],
        axis=1)
    tmp = jnp.where(mask_e > 0.5, s, 0.0)
    sel_e = _topk_mask(tmp, TOPK)
    w = scores * sel_e
    return w / jnp.sum(w, axis=-1, keepdims=True)


def _moe_body(x_ref, gw_ref, bias_ref, wgu_ref, sgu_ref, wdt_ref, out_ref):
    x = x_ref[...]
    logits = jax.lax.dot_general(
        x, gw_ref[...], (((1,), (1,)), ((), ())),
        preferred_element_type=jnp.float32)
    w_full = _routing_weights(logits, bias_ref[...])

    gu_all = jax.lax.dot_general(
        x, wgu_ref[...], (((1,), (1,)), ((), ())),
        preferred_element_type=jnp.float32)
    sgu = jax.lax.dot_general(
        x, sgu_ref[...], (((1,), (1,)), ((), ())),
        preferred_element_type=jnp.float32)

    h_parts = []
    for e in range(E):
        gate = gu_all[:, e * 2 * DFF:e * 2 * DFF + DFF]
        up = gu_all[:, e * 2 * DFF + DFF:(e + 1) * 2 * DFF]
        we = w_full[:, e:e + 1] * RSF
        h_parts.append((gate * jax.nn.sigmoid(gate) * up * we)
                       .astype(jnp.bfloat16))
    sgate = sgu[:, :DFF]
    sup = sgu[:, DFF:]
    h_parts.append((sgate * jax.nn.sigmoid(sgate) * sup).astype(jnp.bfloat16))
    h_all = jnp.concatenate(h_parts, axis=1)

    out_ref[...] = jax.lax.dot_general(
        h_all, wdt_ref[...], (((1,), (0,)), ((), ())),
        preferred_element_type=jnp.float32)


@jax.jit
def kernel(hidden_states, gate_W, e_score_correction_bias, We_gate_up,
           We_down, Ws_gate_up, Ws_down):
    bias2d = e_score_correction_bias.reshape(1, E)
    wgu2d = We_gate_up.reshape(E * 2 * DFF, H)
    wdt = jnp.concatenate(
        [We_down.swapaxes(1, 2).reshape(E * DFF, H), Ws_down.T],
        axis=0).astype(jnp.bfloat16)

    grid = (T // BT,)
    return pl.pallas_call(
        _moe_body,
        grid=grid,
        in_specs=[
            pl.BlockSpec((BT, H), lambda i: (i, 0)),
            pl.BlockSpec((E, H), lambda i: (0, 0)),
            pl.BlockSpec((1, E), lambda i: (0, 0)),
            pl.BlockSpec((E * 2 * DFF, H), lambda i: (0, 0)),
            pl.BlockSpec((2 * DFF, H), lambda i: (0, 0)),
            pl.BlockSpec(((E + 1) * DFF, H), lambda i: (0, 0)),
        ],
        out_specs=pl.BlockSpec((BT, H), lambda i: (i, 0)),
        out_shape=jax.ShapeDtypeStruct((T, H), jnp.float32),
        compiler_params=pltpu.CompilerParams(
            dimension_semantics=("arbitrary",),
        ),
    )(hidden_states, gate_W, bias2d, wgu2d, Ws_gate_up, wdt)


# speedup vs baseline: 2.0235x; 1.2402x over previous
import jax
import jax.numpy as jnp
from jax.experimental import pallas as pl
from jax.experimental.pallas import tpu as pltpu

T = 2048
H = 768
E = 8
TOPK = 2
DFF = 384
NG = 4
TG = 2
RSF = 2.5

BT = 256


def _topk_mask_t(x, k):
    n = x.shape[0]
    rank_rows = []
    for j in range(n):
        xj = x[j:j + 1, :]
        beats = (x > xj).astype(jnp.float32)
        rank_j = jnp.sum(beats, axis=0, keepdims=True)
        if j > 0:
            ties = (x[:j, :] == xj).astype(jnp.float32)
            rank_j = rank_j + jnp.sum(ties, axis=0, keepdims=True)
        rank_rows.append(rank_j)
    rank = jnp.concatenate(rank_rows, axis=0)
    return (rank < k).astype(jnp.float32)


def _routing_weights_t(logits_t, bias_col):
    scores = jax.nn.sigmoid(logits_t)
    s = scores + bias_col
    g = jnp.concatenate(
        [s[2 * j:2 * j + 1, :] + s[2 * j + 1:2 * j + 2, :]
         for j in range(NG)], axis=0)
    sel_g = _topk_mask_t(g, TG)
    mask_e = jnp.concatenate(
        [sel_g[e // (E // NG):e // (E // NG) + 1, :] for e in range(E)],
        axis=0)
    tmp = jnp.where(mask_e > 0.5, s, 0.0)
    sel_e = _topk_mask_t(tmp, TOPK)
    w = scores * sel_e
    return w / jnp.sum(w, axis=0, keepdims=True)


def _moe_body(x_ref, gw_ref, bias_ref, wgu_ref, sgu_ref, wdt_ref, out_ref):
    x = x_ref[...]
    logits_t = jax.lax.dot_general(
        gw_ref[...], x, (((1,), (1,)), ((), ())),
        preferred_element_type=jnp.float32)
    gu_all = jax.lax.dot_general(
        x, wgu_ref[...], (((1,), (1,)), ((), ())),
        preferred_element_type=jnp.float32)
    sgu = jax.lax.dot_general(
        x, sgu_ref[...], (((1,), (1,)), ((), ())),
        preferred_element_type=jnp.float32)

    w_t = _routing_weights_t(logits_t, bias_ref[...])
    w_full = w_t.T * RSF

    h_parts = []
    for e in range(E):
        gate = gu_all[:, e * 2 * DFF:e * 2 * DFF + DFF]
        up = gu_all[:, e * 2 * DFF + DFF:(e + 1) * 2 * DFF]
        h_parts.append((gate * jax.nn.sigmoid(gate) * up
                        * w_full[:, e:e + 1]).astype(jnp.bfloat16))
    sgate = sgu[:, :DFF]
    sup = sgu[:, DFF:]
    h_parts.append((sgate * jax.nn.sigmoid(sgate) * sup).astype(jnp.bfloat16))
    h_all = jnp.concatenate(h_parts, axis=1)

    out_ref[...] = jax.lax.dot_general(
        h_all, wdt_ref[...], (((1,), (0,)), ((), ())),
        preferred_element_type=jnp.float32)


@jax.jit
def kernel(hidden_states, gate_W, e_score_correction_bias, We_gate_up,
           We_down, Ws_gate_up, Ws_down):
    bias_col = e_score_correction_bias.reshape(E, 1)
    wgu2d = We_gate_up.reshape(E * 2 * DFF, H)
    wdt = jnp.concatenate(
        [We_down.swapaxes(1, 2).reshape(E * DFF, H), Ws_down.T],
        axis=0).astype(jnp.bfloat16)

    grid = (T // BT,)
    return pl.pallas_call(
        _moe_body,
        grid=grid,
        in_specs=[
            pl.BlockSpec((BT, H), lambda i: (i, 0)),
            pl.BlockSpec((E, H), lambda i: (0, 0)),
            pl.BlockSpec((E, 1), lambda i: (0, 0)),
            pl.BlockSpec((E * 2 * DFF, H), lambda i: (0, 0)),
            pl.BlockSpec((2 * DFF, H), lambda i: (0, 0)),
            pl.BlockSpec(((E + 1) * DFF, H), lambda i: (0, 0)),
        ],
        out_specs=pl.BlockSpec((BT, H), lambda i: (i, 0)),
        out_shape=jax.ShapeDtypeStruct((T, H), jnp.float32),
        compiler_params=pltpu.CompilerParams(
            dimension_semantics=("arbitrary",),
        ),
    )(hidden_states, gate_W, bias_col, wgu2d, Ws_gate_up, wdt)
